# Initial kernel scaffold; baseline (speedup 1.0000x reference)
import jax
import jax.numpy as jnp
from jax.experimental import pallas as pl


def kernel(x, edge_index, W1l, W1r, b1, ln1_g, ln1_b, Wg, a_src, a_dst, bg, ln2_g, ln2_b, W3l, W3r, b3, Wc1, bc1, Wc2, bc2):
    raise NotImplementedError("write your pallas kernel here")


# speedup vs baseline: 15.6009x; 15.6009x over previous
import functools
import jax
import jax.numpy as jnp
from jax import lax
from jax.experimental import pallas as pl
from jax.experimental.pallas import tpu as pltpu
from jax.experimental.pallas import tpu_sc as plsc

N = 10000
E = 320000
D = 128
H = 4

NC = 2
NS = 16
L = 16
NW = NC * NS
EPW = E // NW
CH = 80
NCHUNK = EPW // CH
_MESH = plsc.VectorSubcoreMesh(core_axis_name="c", subcore_axis_name="s")

_BAND = [(t * 624, 640 if t == NS - 1 else 624) for t in range(NS)]


def _banded_copy(sid, make_src, make_dst):
    for t, (off, sz) in enumerate(_BAND):
        @pl.when(sid == t)
        def _():
            pltpu.sync_copy(make_src(off, sz), make_dst(off, sz))


def _zero_vmem_1d(ref, n):
    z = jnp.zeros((L,), jnp.float32)

    def body(i, _):
        ref[pl.ds(i * L, L)] = z
        return 0

    lax.fori_loop(0, n // L, body, 0)


def _make_segsum(with_deg):
    outs = [jax.ShapeDtypeStruct((NC, N, D), jnp.float32)]
    if with_deg:
        outs.append(jax.ShapeDtypeStruct((NW, N), jnp.float32))

    scratch = [
        pltpu.VMEM((CH,), jnp.int32),
        pltpu.VMEM((CH,), jnp.int32),
        pltpu.VMEM((CH, D), jnp.float32),
        pltpu.VMEM((N,), jnp.float32),
        pltpu.VMEM_SHARED((N, D), jnp.float32),
        pltpu.SemaphoreType.DMA,
    ]

    @functools.partial(
        pl.kernel, mesh=_MESH, out_type=tuple(outs), scratch_types=scratch,
        name="sc_segsum_deg" if with_deg else "sc_segsum",
        compiler_params=pltpu.CompilerParams(needs_layout_passes=False),
    )
    def k(x_hbm, src_hbm, dst_hbm, z_hbm, *refs):
        if with_deg:
            acc_out, deg_out = refs[0], refs[1]
            refs = refs[2:]
        else:
            acc_out = refs[0]
            refs = refs[1:]
        sidx_v, didx_v, rows_v, deg_v, acc_sh, sem = refs

        cid = lax.axis_index("c")
        sid = lax.axis_index("s")
        wid = sid * NC + cid

        _banded_copy(sid, lambda o, s: z_hbm.at[pl.ds(o, s)],
                     lambda o, s: acc_sh.at[pl.ds(o, s)])
        if with_deg:
            _zero_vmem_1d(deg_v, N)
        plsc.subcore_barrier()

        ones = jnp.ones((L,), jnp.float32)

        def chunk(g, _):
            base = wid * EPW + g * CH
            pltpu.sync_copy(src_hbm.at[pl.ds(base, CH)], sidx_v)
            pltpu.sync_copy(dst_hbm.at[pl.ds(base, CH)], didx_v)
            cp = pltpu.async_copy(x_hbm.at[sidx_v], rows_v, sem)
            if with_deg:
                for i in range(CH // L):
                    d16 = didx_v[pl.ds(i * L, L)]
                    plsc.addupdate_scatter(deg_v, [d16], ones)
            cp.wait()
            pltpu.sync_copy(rows_v, acc_sh.at[didx_v], add=True)
            return 0

        lax.fori_loop(0, NCHUNK, chunk, 0)

        plsc.subcore_barrier()
        _banded_copy(sid, lambda o, s: acc_sh.at[pl.ds(o, s)],
                     lambda o, s: acc_out.at[cid, pl.ds(o, s), :])
        if with_deg:
            pltpu.sync_copy(deg_v, deg_out.at[wid])

    return k


_segsum_deg = _make_segsum(True)
_segsum = _make_segsum(False)


@functools.partial(
    pl.kernel, mesh=_MESH,
    out_type=(
        jax.ShapeDtypeStruct((E * H,), jnp.float32),
        jax.ShapeDtypeStruct((NW, N * H), jnp.float32),
    ),
    scratch_types=[
        pltpu.VMEM((N * H,), jnp.float32),
        pltpu.VMEM((N * H,), jnp.float32),
        pltpu.VMEM((N * H,), jnp.float32),
        pltpu.VMEM((CH,), jnp.int32),
        pltpu.VMEM((CH,), jnp.int32),
        pltpu.VMEM((CH * H,), jnp.float32),
    ],
    name="sc_gat_edge",
    compiler_params=pltpu.CompilerParams(needs_layout_passes=False),
)
def _gat_edge(als_hbm, ald_hbm, src_hbm, dst_hbm, ee_out, den_out,
              als_v, ald_v, den_v, sidx_v, didx_v, eec_v):
    cid = lax.axis_index("c")
    sid = lax.axis_index("s")
    wid = sid * NC + cid

    pltpu.sync_copy(als_hbm, als_v)
    pltpu.sync_copy(ald_hbm, ald_v)
    _zero_vmem_1d(den_v, N * H)

    iota = lax.broadcasted_iota(jnp.int32, (L,), 0)

    def chunk(g, _):
        base = wid * EPW + g * CH
        pltpu.sync_copy(src_hbm.at[pl.ds(base, CH)], sidx_v)
        pltpu.sync_copy(dst_hbm.at[pl.ds(base, CH)], didx_v)
        for i in range(CH // L):
            s16 = sidx_v[pl.ds(i * L, L)] * H
            d16 = didx_v[pl.ds(i * L, L)] * H
            for h in range(H):
                e = (plsc.load_gather(als_v, [s16 + h])
                     + plsc.load_gather(ald_v, [d16 + h]))
                e = jnp.where(e >= 0.0, e, 0.2 * e)
                ee = jnp.exp(e)
                plsc.store_scatter(eec_v, [iota * H + (i * L * H + h)], ee)
                plsc.addupdate_scatter(den_v, [d16 + h], ee)
        pltpu.sync_copy(eec_v, ee_out.at[pl.ds(base * H, CH * H)])
        return 0

    lax.fori_loop(0, NCHUNK, chunk, 0)
    pltpu.sync_copy(den_v, den_out.at[wid])


@functools.partial(
    pl.kernel, mesh=_MESH,
    out_type=jax.ShapeDtypeStruct((E * H,), jnp.float32),
    scratch_types=[
        pltpu.VMEM((N * H,), jnp.float32),
        pltpu.VMEM((CH,), jnp.int32),
        pltpu.VMEM((CH * H,), jnp.float32),
        pltpu.VMEM((CH * H,), jnp.float32),
    ],
    name="sc_gat_alpha",
    compiler_params=pltpu.CompilerParams(needs_layout_passes=False),
)
def _gat_alpha(ee_hbm, rden_hbm, dst_hbm, w_out, rden_v, didx_v, eec_v, wc_v):
    cid = lax.axis_index("c")
    sid = lax.axis_index("s")
    wid = sid * NC + cid

    pltpu.sync_copy(rden_hbm, rden_v)
    iota = lax.broadcasted_iota(jnp.int32, (L,), 0)

    def chunk(g, _):
        base = wid * EPW + g * CH
        pltpu.sync_copy(dst_hbm.at[pl.ds(base, CH)], didx_v)
        pltpu.sync_copy(ee_hbm.at[pl.ds(base * H, CH * H)], eec_v)
        for i in range(CH // L):
            d16 = didx_v[pl.ds(i * L, L)] * H
            for h in range(H):
                pos = iota * H + (i * L * H + h)
                ee16 = plsc.load_gather(eec_v, [pos])
                r16 = plsc.load_gather(rden_v, [d16 + h])
                plsc.store_scatter(wc_v, [pos], ee16 * r16)
        pltpu.sync_copy(wc_v, w_out.at[pl.ds(base * H, CH * H)])
        return 0

    lax.fori_loop(0, NCHUNK, chunk, 0)


CH3 = 40
NCHUNK3 = EPW // CH3

@functools.partial(
    pl.kernel, mesh=_MESH,
    out_type=jax.ShapeDtypeStruct((NC, N, D), jnp.float32),
    scratch_types=[
        pltpu.VMEM((CH3,), jnp.int32),
        pltpu.VMEM((CH3,), jnp.int32),
        pltpu.VMEM((CH3 * H,), jnp.float32),
        pltpu.VMEM((CH3, H * D), jnp.float32),
        pltpu.VMEM((CH3, D), jnp.float32),
        pltpu.VMEM_SHARED((N, D), jnp.float32),
        pltpu.SemaphoreType.DMA,
    ],
    name="sc_gat_agg",
    compiler_params=pltpu.CompilerParams(needs_layout_passes=False),
)
def _gat_agg(hg_hbm, w_hbm, src_hbm, dst_hbm, z_hbm, acc_out,
             sidx_v, didx_v, w_v, rows_v, msg_v, acc_sh, sem):
    cid = lax.axis_index("c")
    sid = lax.axis_index("s")
    wid = sid * NC + cid

    _banded_copy(sid, lambda o, s: z_hbm.at[pl.ds(o, s)],
                 lambda o, s: acc_sh.at[pl.ds(o, s)])
    plsc.subcore_barrier()

    def chunk(g, _):
        base = wid * EPW + g * CH3
        pltpu.sync_copy(src_hbm.at[pl.ds(base, CH3)], sidx_v)
        pltpu.sync_copy(dst_hbm.at[pl.ds(base, CH3)], didx_v)
        cp = pltpu.async_copy(hg_hbm.at[sidx_v], rows_v, sem)
        pltpu.sync_copy(w_hbm.at[pl.ds(base * H, CH3 * H)], w_v)
        cp.wait()

        def edge(i, _):
            wb0 = plsc.load_gather(w_v, [jnp.broadcast_to(i * H + 0, (L,))])
            wb1 = plsc.load_gather(w_v, [jnp.broadcast_to(i * H + 1, (L,))])
            wb2 = plsc.load_gather(w_v, [jnp.broadcast_to(i * H + 2, (L,))])
            wb3 = plsc.load_gather(w_v, [jnp.broadcast_to(i * H + 3, (L,))])
            for j in range(D // L):
                m = wb0 * rows_v[i, pl.ds(j * L, L)]
                m = m + wb1 * rows_v[i, pl.ds(D + j * L, L)]
                m = m + wb2 * rows_v[i, pl.ds(2 * D + j * L, L)]
                m = m + wb3 * rows_v[i, pl.ds(3 * D + j * L, L)]
                msg_v[i, pl.ds(j * L, L)] = m
            return 0

        lax.fori_loop(0, CH3, edge, 0)
        pltpu.sync_copy(msg_v, acc_sh.at[didx_v], add=True)
        return 0

    lax.fori_loop(0, NCHUNK3, chunk, 0)

    plsc.subcore_barrier()
    _banded_copy(sid, lambda o, s: acc_sh.at[pl.ds(o, s)],
                 lambda o, s: acc_out.at[cid, pl.ds(o, s), :])


RB = 1000
GRID = N // RB


def _ln(h, g, b):
    m = jnp.mean(h, axis=-1, keepdims=True)
    v = jnp.mean((h - m) * (h - m), axis=-1, keepdims=True)
    return (h - m) * jax.lax.rsqrt(v + 1e-5) * g + b


def _elu(h):
    return jnp.where(h > 0.0, h, jnp.exp(jnp.minimum(h, 0.0)) - 1.0)


def _tc_b_body(x_ref, acc_ref, degp_ref, W1l_ref, W1r_ref, b1_ref, g1_ref,
               bb1_ref, Wg_ref, as_ref, ad_ref,
               hg_ref, als_ref, ald_ref, deg_ref):
    deg = jnp.maximum(jnp.sum(degp_ref[...], axis=0), 1.0)
    deg_ref[...] = deg
    agg = (acc_ref[0] + acc_ref[1]) / deg
    s1 = agg @ W1l_ref[...] + x_ref[...] @ W1r_ref[...] + b1_ref[...][None]
    h1 = _elu(_ln(s1, g1_ref[...][None], bb1_ref[...][None]))
    hg = h1 @ Wg_ref[...]
    hg_ref[...] = hg
    als_ref[...] = hg @ as_ref[...]
    ald_ref[...] = hg @ ad_ref[...]


def _tc_b(x, acc, degp, W1l, W1r, b1, g1, bb1, Wg2d, As2, Ad2):
    return pl.pallas_call(
        _tc_b_body,
        grid=(GRID,),
        in_specs=[
            pl.BlockSpec((RB, D), lambda i: (i, 0)),
            pl.BlockSpec((NC, RB, D), lambda i: (0, i, 0)),
            pl.BlockSpec((NW, RB, 1), lambda i: (0, i, 0)),
            pl.BlockSpec((D, D), lambda i: (0, 0)),
            pl.BlockSpec((D, D), lambda i: (0, 0)),
            pl.BlockSpec((D,), lambda i: (0,)),
            pl.BlockSpec((D,), lambda i: (0,)),
            pl.BlockSpec((D,), lambda i: (0,)),
            pl.BlockSpec((D, H * D), lambda i: (0, 0)),
            pl.BlockSpec((H * D, H), lambda i: (0, 0)),
            pl.BlockSpec((H * D, H), lambda i: (0, 0)),
        ],
        out_specs=[
            pl.BlockSpec((RB, H * D), lambda i: (i, 0)),
            pl.BlockSpec((RB, H), lambda i: (i, 0)),
            pl.BlockSpec((RB, H), lambda i: (i, 0)),
            pl.BlockSpec((RB, 1), lambda i: (i, 0)),
        ],
        out_shape=[
            jax.ShapeDtypeStruct((N, H * D), jnp.float32),
            jax.ShapeDtypeStruct((N, H), jnp.float32),
            jax.ShapeDtypeStruct((N, H), jnp.float32),
            jax.ShapeDtypeStruct((N, 1), jnp.float32),
        ],
    )(x, acc, degp, W1l, W1r, b1, g1, bb1, Wg2d, As2, Ad2)


def _tc_rden_body(den_ref, out_ref):
    s = jnp.sum(den_ref[...], axis=0)
    out_ref[...] = 1.0 / jnp.maximum(s, 1e-16)


def _tc_rden(denp):
    return pl.pallas_call(
        _tc_rden_body,
        grid=(GRID,),
        in_specs=[pl.BlockSpec((NW, RB, H), lambda i: (0, i, 0))],
        out_specs=pl.BlockSpec((RB, H), lambda i: (i, 0)),
        out_shape=jax.ShapeDtypeStruct((N, H), jnp.float32),
    )(denp)


def _tc_f_body(acc_ref, bg_ref, g2_ref, b2_ref, out_ref):
    gout = (acc_ref[0] + acc_ref[1]) * (1.0 / H) + bg_ref[...][None]
    out_ref[...] = _elu(_ln(gout, g2_ref[...][None], b2_ref[...][None]))


def _tc_f(acc, bg, g2, b2):
    return pl.pallas_call(
        _tc_f_body,
        grid=(GRID,),
        in_specs=[
            pl.BlockSpec((NC, RB, D), lambda i: (0, i, 0)),
            pl.BlockSpec((D,), lambda i: (0,)),
            pl.BlockSpec((D,), lambda i: (0,)),
            pl.BlockSpec((D,), lambda i: (0,)),
        ],
        out_specs=pl.BlockSpec((RB, D), lambda i: (i, 0)),
        out_shape=jax.ShapeDtypeStruct((N, D), jnp.float32),
    )(acc, bg, g2, b2)


def _tc_h_body(acc_ref, deg_ref, h2_ref, W3l_ref, W3r_ref, b3_ref,
               Wc1_ref, bc1_ref, Wc2_ref, bc2_ref, out_ref):
    agg = (acc_ref[0] + acc_ref[1]) / deg_ref[...]
    o3 = agg @ W3l_ref[...] + h2_ref[...] @ W3r_ref[...] + b3_ref[...][None]
    z = jnp.maximum(o3 @ Wc1_ref[...] + bc1_ref[...][None], 0.0)
    out_ref[...] = z @ Wc2_ref[...] + bc2_ref[...][None]


def _tc_h(acc, deg, h2, W3l, W3r, b3, Wc1, bc1, Wc2, bc2):
    return pl.pallas_call(
        _tc_h_body,
        grid=(GRID,),
        in_specs=[
            pl.BlockSpec((NC, RB, D), lambda i: (0, i, 0)),
            pl.BlockSpec((RB, 1), lambda i: (i, 0)),
            pl.BlockSpec((RB, D), lambda i: (i, 0)),
            pl.BlockSpec((D, D), lambda i: (0, 0)),
            pl.BlockSpec((D, D), lambda i: (0, 0)),
            pl.BlockSpec((D,), lambda i: (0,)),
            pl.BlockSpec((D, 64), lambda i: (0, 0)),
            pl.BlockSpec((64,), lambda i: (0,)),
            pl.BlockSpec((64, 2), lambda i: (0, 0)),
            pl.BlockSpec((2,), lambda i: (0,)),
        ],
        out_specs=pl.BlockSpec((RB, 2), lambda i: (i, 0)),
        out_shape=jax.ShapeDtypeStruct((N, 2), jnp.float32),
    )(acc, deg, h2, W3l, W3r, b3, Wc1, bc1, Wc2, bc2)


def kernel(x, edge_index, W1l, W1r, b1, ln1_g, ln1_b, Wg, a_src, a_dst, bg,
           ln2_g, ln2_b, W3l, W3r, b3, Wc1, bc1, Wc2, bc2):
    src = edge_index[0]
    dst = edge_index[1]
    zND = jnp.zeros((N, D), jnp.float32)

    eye = jnp.repeat(jnp.eye(H, dtype=jnp.float32), D, axis=0)
    As2 = eye * a_src.reshape(H * D)[:, None]
    Ad2 = eye * a_dst.reshape(H * D)[:, None]

    acc1, degp = _segsum_deg(x, src, dst, zND)
    hg, als, ald, deg = _tc_b(x, acc1, degp.reshape(NW, N, 1), W1l, W1r, b1, ln1_g, ln1_b,
                              Wg.reshape(D, H * D), As2, Ad2)
    ee, denp = _gat_edge(als.reshape(N * H), ald.reshape(N * H), src, dst)
    rden = _tc_rden(denp.reshape(NW, N, H))
    w = _gat_alpha(ee, rden.reshape(N * H), dst)
    gacc = _gat_agg(hg, w, src, dst, zND)
    h2 = _tc_f(gacc, bg, ln2_g, ln2_b)
    acc3, = _segsum(h2, src, dst, zND)
    return _tc_h(acc3, deg, h2, W3l, W3r, b3, Wc1, bc1, Wc2, bc2)


# speedup vs baseline: 19.1768x; 1.2292x over previous
import functools
import jax
import jax.numpy as jnp
from jax import lax
from jax.experimental import pallas as pl
from jax.experimental.pallas import tpu as pltpu
from jax.experimental.pallas import tpu_sc as plsc

N = 10000
E = 320000
D = 128
H = 4

NC = 2
NS = 16
L = 16
NW = NC * NS
EPW = E // NW
CH = 80
NCHUNK = EPW // CH
_MESH = plsc.VectorSubcoreMesh(core_axis_name="c", subcore_axis_name="s")

_BAND = [(t * 624, 640 if t == NS - 1 else 624) for t in range(NS)]


def _banded_copy(sid, make_src, make_dst):
    for t, (off, sz) in enumerate(_BAND):
        @pl.when(sid == t)
        def _():
            pltpu.sync_copy(make_src(off, sz), make_dst(off, sz))


def _zero_vmem_1d(ref, n):
    z = jnp.zeros((L,), jnp.float32)

    def body(i, _):
        ref[pl.ds(i * L, L)] = z
        return 0

    lax.fori_loop(0, n // L, body, 0)


def _make_segsum(with_deg):
    outs = [jax.ShapeDtypeStruct((NC, N, D), jnp.float32)]
    if with_deg:
        outs.append(jax.ShapeDtypeStruct((NW, N), jnp.float32))

    scratch = [
        pltpu.VMEM((CH,), jnp.int32),
        pltpu.VMEM((CH,), jnp.int32),
        pltpu.VMEM((CH,), jnp.int32),
        pltpu.VMEM((CH,), jnp.int32),
        pltpu.VMEM((CH, D), jnp.float32),
        pltpu.VMEM((CH, D), jnp.float32),
        pltpu.VMEM((N,), jnp.float32),
        pltpu.VMEM_SHARED((N, D), jnp.float32),
        pltpu.SemaphoreType.DMA,
        pltpu.SemaphoreType.DMA,
    ]

    @functools.partial(
        pl.kernel, mesh=_MESH, out_type=tuple(outs), scratch_types=scratch,
        name="sc_segsum_deg" if with_deg else "sc_segsum",
        compiler_params=pltpu.CompilerParams(needs_layout_passes=False),
    )
    def k(x_hbm, src_hbm, dst_hbm, z_hbm, *refs):
        if with_deg:
            acc_out, deg_out = refs[0], refs[1]
            refs = refs[2:]
        else:
            acc_out = refs[0]
            refs = refs[1:]
        (sidx0, sidx1, didx0, didx1, rows0, rows1, deg_v, acc_sh,
         sem0, sem1) = refs

        cid = lax.axis_index("c")
        sid = lax.axis_index("s")
        wid = sid * NC + cid
        ebase = wid * EPW

        _banded_copy(sid, lambda o, s: z_hbm.at[pl.ds(o, s)],
                     lambda o, s: acc_sh.at[pl.ds(o, s)])
        if with_deg:
            _zero_vmem_1d(deg_v, N)
        plsc.subcore_barrier()

        ones = jnp.ones((L,), jnp.float32)
        bufs = ((sidx0, didx0, rows0, sem0), (sidx1, didx1, rows1, sem1))

        def start(g, b):
            sidx, didx, rows, sem = bufs[b]
            base = ebase + g * CH
            pltpu.sync_copy(src_hbm.at[pl.ds(base, CH)], sidx)
            pltpu.sync_copy(dst_hbm.at[pl.ds(base, CH)], didx)
            return pltpu.async_copy(x_hbm.at[sidx], rows, sem)

        def finish(b):
            sidx, didx, rows, sem = bufs[b]
            pltpu.make_async_copy(x_hbm.at[sidx], rows, sem).wait()
            if with_deg:
                for i in range(CH // L):
                    d16 = didx[pl.ds(i * L, L)]
                    plsc.addupdate_scatter(deg_v, [d16], ones)
            pltpu.sync_copy(rows, acc_sh.at[didx], add=True)

        start(0, 0)

        def pair(p, _):
            start(2 * p + 1, 1)
            finish(0)

            @pl.when(2 * p + 2 < NCHUNK)
            def _():
                start(2 * p + 2, 0)
            finish(1)
            return 0

        lax.fori_loop(0, NCHUNK // 2, pair, 0)
        if NCHUNK % 2 == 1:
            finish(0)

        plsc.subcore_barrier()
        _banded_copy(sid, lambda o, s: acc_sh.at[pl.ds(o, s)],
                     lambda o, s: acc_out.at[cid, pl.ds(o, s), :])
        if with_deg:
            pltpu.sync_copy(deg_v, deg_out.at[wid])

    return k


_segsum_deg = _make_segsum(True)
_segsum = _make_segsum(False)


@functools.partial(
    pl.kernel, mesh=_MESH,
    out_type=(
        jax.ShapeDtypeStruct((E * H,), jnp.float32),
        jax.ShapeDtypeStruct((NW, N * H), jnp.float32),
    ),
    scratch_types=[
        pltpu.VMEM((N * H,), jnp.float32),
        pltpu.VMEM((N * H,), jnp.float32),
        pltpu.VMEM((N * H,), jnp.float32),
        pltpu.VMEM((CH,), jnp.int32),
        pltpu.VMEM((CH,), jnp.int32),
        pltpu.VMEM((CH * H,), jnp.float32),
    ],
    name="sc_gat_edge",
    compiler_params=pltpu.CompilerParams(needs_layout_passes=False),
)
def _gat_edge(als_hbm, ald_hbm, src_hbm, dst_hbm, ee_out, den_out,
              als_v, ald_v, den_v, sidx_v, didx_v, eec_v):
    cid = lax.axis_index("c")
    sid = lax.axis_index("s")
    wid = sid * NC + cid

    pltpu.sync_copy(als_hbm, als_v)
    pltpu.sync_copy(ald_hbm, ald_v)
    _zero_vmem_1d(den_v, N * H)

    iota = lax.broadcasted_iota(jnp.int32, (L,), 0)

    def chunk(g, _):
        base = wid * EPW + g * CH
        pltpu.sync_copy(src_hbm.at[pl.ds(base, CH)], sidx_v)
        pltpu.sync_copy(dst_hbm.at[pl.ds(base, CH)], didx_v)
        for i in range(CH // L):
            s16 = sidx_v[pl.ds(i * L, L)] * H
            d16 = didx_v[pl.ds(i * L, L)] * H
            for h in range(H):
                e = (plsc.load_gather(als_v, [s16 + h])
                     + plsc.load_gather(ald_v, [d16 + h]))
                e = jnp.where(e >= 0.0, e, 0.2 * e)
                ee = jnp.exp(e)
                plsc.store_scatter(eec_v, [iota * H + (i * L * H + h)], ee)
                plsc.addupdate_scatter(den_v, [d16 + h], ee)
        pltpu.sync_copy(eec_v, ee_out.at[pl.ds(base * H, CH * H)])
        return 0

    lax.fori_loop(0, NCHUNK, chunk, 0)
    pltpu.sync_copy(den_v, den_out.at[wid])


@functools.partial(
    pl.kernel, mesh=_MESH,
    out_type=jax.ShapeDtypeStruct((E * H,), jnp.float32),
    scratch_types=[
        pltpu.VMEM((N * H,), jnp.float32),
        pltpu.VMEM((CH,), jnp.int32),
        pltpu.VMEM((CH * H,), jnp.float32),
        pltpu.VMEM((CH * H,), jnp.float32),
    ],
    name="sc_gat_alpha",
    compiler_params=pltpu.CompilerParams(needs_layout_passes=False),
)
def _gat_alpha(ee_hbm, rden_hbm, dst_hbm, w_out, rden_v, didx_v, eec_v, wc_v):
    cid = lax.axis_index("c")
    sid = lax.axis_index("s")
    wid = sid * NC + cid

    pltpu.sync_copy(rden_hbm, rden_v)
    iota = lax.broadcasted_iota(jnp.int32, (L,), 0)

    def chunk(g, _):
        base = wid * EPW + g * CH
        pltpu.sync_copy(dst_hbm.at[pl.ds(base, CH)], didx_v)
        pltpu.sync_copy(ee_hbm.at[pl.ds(base * H, CH * H)], eec_v)
        for i in range(CH // L):
            d16 = didx_v[pl.ds(i * L, L)] * H
            for h in range(H):
                pos = iota * H + (i * L * H + h)
                ee16 = plsc.load_gather(eec_v, [pos])
                r16 = plsc.load_gather(rden_v, [d16 + h])
                plsc.store_scatter(wc_v, [pos], ee16 * r16)
        pltpu.sync_copy(wc_v, w_out.at[pl.ds(base * H, CH * H)])
        return 0

    lax.fori_loop(0, NCHUNK, chunk, 0)


CH3 = 40
NCHUNK3 = EPW // CH3

@functools.partial(
    pl.kernel, mesh=_MESH,
    out_type=jax.ShapeDtypeStruct((NC, N, D), jnp.float32),
    scratch_types=[
        pltpu.VMEM((CH3,), jnp.int32),
        pltpu.VMEM((CH3,), jnp.int32),
        pltpu.VMEM((CH3,), jnp.int32),
        pltpu.VMEM((CH3,), jnp.int32),
        pltpu.VMEM((CH3 * H,), jnp.float32),
        pltpu.VMEM((CH3 * H,), jnp.float32),
        pltpu.VMEM((CH3, H * D), jnp.float32),
        pltpu.VMEM((CH3, H * D), jnp.float32),
        pltpu.VMEM((CH3, D), jnp.float32),
        pltpu.VMEM_SHARED((N, D), jnp.float32),
        pltpu.SemaphoreType.DMA,
        pltpu.SemaphoreType.DMA,
    ],
    name="sc_gat_agg",
    compiler_params=pltpu.CompilerParams(needs_layout_passes=False),
)
def _gat_agg(hg_hbm, w_hbm, src_hbm, dst_hbm, z_hbm, acc_out,
             sidx0, sidx1, didx0, didx1, w0, w1, rows0, rows1, msg_v,
             acc_sh, sem0, sem1):
    cid = lax.axis_index("c")
    sid = lax.axis_index("s")
    wid = sid * NC + cid
    ebase = wid * EPW

    _banded_copy(sid, lambda o, s: z_hbm.at[pl.ds(o, s)],
                 lambda o, s: acc_sh.at[pl.ds(o, s)])
    plsc.subcore_barrier()

    bufs = ((sidx0, didx0, w0, rows0, sem0), (sidx1, didx1, w1, rows1, sem1))

    def start(g, b):
        sidx, didx, w_v, rows, sem = bufs[b]
        base = ebase + g * CH3
        pltpu.sync_copy(src_hbm.at[pl.ds(base, CH3)], sidx)
        pltpu.sync_copy(dst_hbm.at[pl.ds(base, CH3)], didx)
        pltpu.sync_copy(w_hbm.at[pl.ds(base * H, CH3 * H)], w_v)
        pltpu.async_copy(hg_hbm.at[sidx], rows, sem)

    def finish(b):
        sidx, didx, w_v, rows, sem = bufs[b]
        pltpu.make_async_copy(hg_hbm.at[sidx], rows, sem).wait()

        def edge4(q, _):
            w16 = w_v[pl.ds(q * (4 * H), 4 * H)]
            for k in range(4):
                i = q * 4 + k
                wb = [w16.at[jnp.broadcast_to(k * H + h, (L,))]
                      .get(mode="promise_in_bounds") for h in range(H)]
                for j in range(D // L):
                    m = wb[0] * rows[i, pl.ds(j * L, L)]
                    m = m + wb[1] * rows[i, pl.ds(D + j * L, L)]
                    m = m + wb[2] * rows[i, pl.ds(2 * D + j * L, L)]
                    m = m + wb[3] * rows[i, pl.ds(3 * D + j * L, L)]
                    msg_v[i, pl.ds(j * L, L)] = m
            return 0

        lax.fori_loop(0, CH3 // 4, edge4, 0)
        pltpu.sync_copy(msg_v, acc_sh.at[didx], add=True)

    start(0, 0)

    def pair(p, _):
        start(2 * p + 1, 1)
        finish(0)

        @pl.when(2 * p + 2 < NCHUNK3)
        def _():
            start(2 * p + 2, 0)
        finish(1)
        return 0

    lax.fori_loop(0, NCHUNK3 // 2, pair, 0)
    if NCHUNK3 % 2 == 1:
        finish(0)

    plsc.subcore_barrier()
    _banded_copy(sid, lambda o, s: acc_sh.at[pl.ds(o, s)],
                 lambda o, s: acc_out.at[cid, pl.ds(o, s), :])


RB = 1000
GRID = N // RB


def _ln(h, g, b):
    m = jnp.mean(h, axis=-1, keepdims=True)
    v = jnp.mean((h - m) * (h - m), axis=-1, keepdims=True)
    return (h - m) * jax.lax.rsqrt(v + 1e-5) * g + b


def _elu(h):
    return jnp.where(h > 0.0, h, jnp.exp(jnp.minimum(h, 0.0)) - 1.0)


def _tc_b_body(x_ref, acc_ref, degp_ref, W1l_ref, W1r_ref, b1_ref, g1_ref,
               bb1_ref, Wg_ref, as_ref, ad_ref,
               hg_ref, als_ref, ald_ref, deg_ref):
    deg = jnp.maximum(jnp.sum(degp_ref[...], axis=0), 1.0)
    deg_ref[...] = deg
    agg = (acc_ref[0] + acc_ref[1]) / deg
    s1 = agg @ W1l_ref[...] + x_ref[...] @ W1r_ref[...] + b1_ref[...][None]
    h1 = _elu(_ln(s1, g1_ref[...][None], bb1_ref[...][None]))
    hg = h1 @ Wg_ref[...]
    hg_ref[...] = hg
    als_ref[...] = hg @ as_ref[...]
    ald_ref[...] = hg @ ad_ref[...]


def _tc_b(x, acc, degp, W1l, W1r, b1, g1, bb1, Wg2d, As2, Ad2):
    return pl.pallas_call(
        _tc_b_body,
        grid=(GRID,),
        in_specs=[
            pl.BlockSpec((RB, D), lambda i: (i, 0)),
            pl.BlockSpec((NC, RB, D), lambda i: (0, i, 0)),
            pl.BlockSpec((NW, RB, 1), lambda i: (0, i, 0)),
            pl.BlockSpec((D, D), lambda i: (0, 0)),
            pl.BlockSpec((D, D), lambda i: (0, 0)),
            pl.BlockSpec((D,), lambda i: (0,)),
            pl.BlockSpec((D,), lambda i: (0,)),
            pl.BlockSpec((D,), lambda i: (0,)),
            pl.BlockSpec((D, H * D), lambda i: (0, 0)),
            pl.BlockSpec((H * D, H), lambda i: (0, 0)),
            pl.BlockSpec((H * D, H), lambda i: (0, 0)),
        ],
        out_specs=[
            pl.BlockSpec((RB, H * D), lambda i: (i, 0)),
            pl.BlockSpec((RB, H), lambda i: (i, 0)),
            pl.BlockSpec((RB, H), lambda i: (i, 0)),
            pl.BlockSpec((RB, 1), lambda i: (i, 0)),
        ],
        out_shape=[
            jax.ShapeDtypeStruct((N, H * D), jnp.float32),
            jax.ShapeDtypeStruct((N, H), jnp.float32),
            jax.ShapeDtypeStruct((N, H), jnp.float32),
            jax.ShapeDtypeStruct((N, 1), jnp.float32),
        ],
    )(x, acc, degp, W1l, W1r, b1, g1, bb1, Wg2d, As2, Ad2)


def _tc_rden_body(den_ref, out_ref):
    s = jnp.sum(den_ref[...], axis=0)
    out_ref[...] = 1.0 / jnp.maximum(s, 1e-16)


def _tc_rden(denp):
    return pl.pallas_call(
        _tc_rden_body,
        grid=(GRID,),
        in_specs=[pl.BlockSpec((NW, RB, H), lambda i: (0, i, 0))],
        out_specs=pl.BlockSpec((RB, H), lambda i: (i, 0)),
        out_shape=jax.ShapeDtypeStruct((N, H), jnp.float32),
    )(denp)


def _tc_f_body(acc_ref, bg_ref, g2_ref, b2_ref, out_ref):
    gout = (acc_ref[0] + acc_ref[1]) * (1.0 / H) + bg_ref[...][None]
    out_ref[...] = _elu(_ln(gout, g2_ref[...][None], b2_ref[...][None]))


def _tc_f(acc, bg, g2, b2):
    return pl.pallas_call(
        _tc_f_body,
        grid=(GRID,),
        in_specs=[
            pl.BlockSpec((NC, RB, D), lambda i: (0, i, 0)),
            pl.BlockSpec((D,), lambda i: (0,)),
            pl.BlockSpec((D,), lambda i: (0,)),
            pl.BlockSpec((D,), lambda i: (0,)),
        ],
        out_specs=pl.BlockSpec((RB, D), lambda i: (i, 0)),
        out_shape=jax.ShapeDtypeStruct((N, D), jnp.float32),
    )(acc, bg, g2, b2)


def _tc_h_body(acc_ref, deg_ref, h2_ref, W3l_ref, W3r_ref, b3_ref,
               Wc1_ref, bc1_ref, Wc2_ref, bc2_ref, out_ref):
    agg = (acc_ref[0] + acc_ref[1]) / deg_ref[...]
    o3 = agg @ W3l_ref[...] + h2_ref[...] @ W3r_ref[...] + b3_ref[...][None]
    z = jnp.maximum(o3 @ Wc1_ref[...] + bc1_ref[...][None], 0.0)
    out_ref[...] = z @ Wc2_ref[...] + bc2_ref[...][None]


def _tc_h(acc, deg, h2, W3l, W3r, b3, Wc1, bc1, Wc2, bc2):
    return pl.pallas_call(
        _tc_h_body,
        grid=(GRID,),
        in_specs=[
            pl.BlockSpec((NC, RB, D), lambda i: (0, i, 0)),
            pl.BlockSpec((RB, 1), lambda i: (i, 0)),
            pl.BlockSpec((RB, D), lambda i: (i, 0)),
            pl.BlockSpec((D, D), lambda i: (0, 0)),
            pl.BlockSpec((D, D), lambda i: (0, 0)),
            pl.BlockSpec((D,), lambda i: (0,)),
            pl.BlockSpec((D, 64), lambda i: (0, 0)),
            pl.BlockSpec((64,), lambda i: (0,)),
            pl.BlockSpec((64, 2), lambda i: (0, 0)),
            pl.BlockSpec((2,), lambda i: (0,)),
        ],
        out_specs=pl.BlockSpec((RB, 2), lambda i: (i, 0)),
        out_shape=jax.ShapeDtypeStruct((N, 2), jnp.float32),
    )(acc, deg, h2, W3l, W3r, b3, Wc1, bc1, Wc2, bc2)


def kernel(x, edge_index, W1l, W1r, b1, ln1_g, ln1_b, Wg, a_src, a_dst, bg,
           ln2_g, ln2_b, W3l, W3r, b3, Wc1, bc1, Wc2, bc2):
    src = edge_index[0]
    dst = edge_index[1]
    zND = jnp.zeros((N, D), jnp.float32)

    eye = jnp.repeat(jnp.eye(H, dtype=jnp.float32), D, axis=0)
    As2 = eye * a_src.reshape(H * D)[:, None]
    Ad2 = eye * a_dst.reshape(H * D)[:, None]

    acc1, degp = _segsum_deg(x, src, dst, zND)
    hg, als, ald, deg = _tc_b(x, acc1, degp.reshape(NW, N, 1), W1l, W1r, b1, ln1_g, ln1_b,
                              Wg.reshape(D, H * D), As2, Ad2)
    ee, denp = _gat_edge(als.reshape(N * H), ald.reshape(N * H), src, dst)
    rden = _tc_rden(denp.reshape(NW, N, H))
    w = _gat_alpha(ee, rden.reshape(N * H), dst)
    gacc = _gat_agg(hg, w, src, dst, zND)
    h2 = _tc_f(gacc, bg, ln2_g, ln2_b)
    acc3, = _segsum(h2, src, dst, zND)
    return _tc_h(acc3, deg, h2, W3l, W3r, b3, Wc1, bc1, Wc2, bc2)


# speedup vs baseline: 21.8442x; 1.1391x over previous
import functools
import jax
import jax.numpy as jnp
from jax import lax
from jax.experimental import pallas as pl
from jax.experimental.pallas import tpu as pltpu
from jax.experimental.pallas import tpu_sc as plsc

N = 10000
E = 320000
D = 128
H = 4

NC = 2
NS = 16
L = 16
NW = NC * NS
EPW = E // NW
CH = 80
NCHUNK = EPW // CH
_MESH = plsc.VectorSubcoreMesh(core_axis_name="c", subcore_axis_name="s")

_BAND = [(t * 624, 640 if t == NS - 1 else 624) for t in range(NS)]


def _banded_copy(sid, make_src, make_dst):
    for t, (off, sz) in enumerate(_BAND):
        @pl.when(sid == t)
        def _():
            pltpu.sync_copy(make_src(off, sz), make_dst(off, sz))


def _zero_vmem_1d(ref, n):
    z = jnp.zeros((L,), jnp.float32)

    def body(i, _):
        ref[pl.ds(i * L, L)] = z
        return 0

    lax.fori_loop(0, n // L, body, 0)


def _make_segsum(with_deg):
    outs = [jax.ShapeDtypeStruct((NC, N, D), jnp.float32)]
    if with_deg:
        outs.append(jax.ShapeDtypeStruct((NW, N), jnp.float32))

    scratch = [
        pltpu.VMEM((CH,), jnp.int32),
        pltpu.VMEM((CH,), jnp.int32),
        pltpu.VMEM((CH,), jnp.int32),
        pltpu.VMEM((CH,), jnp.int32),
        pltpu.VMEM((CH, D), jnp.float32),
        pltpu.VMEM((CH, D), jnp.float32),
        pltpu.VMEM((N,), jnp.float32),
        pltpu.VMEM_SHARED((N, D), jnp.float32),
        pltpu.SemaphoreType.DMA,
        pltpu.SemaphoreType.DMA,
    ]

    @functools.partial(
        pl.kernel, mesh=_MESH, out_type=tuple(outs), scratch_types=scratch,
        name="sc_segsum_deg" if with_deg else "sc_segsum",
        compiler_params=pltpu.CompilerParams(needs_layout_passes=False),
    )
    def k(x_hbm, src_hbm, dst_hbm, z_hbm, *refs):
        if with_deg:
            acc_out, deg_out = refs[0], refs[1]
            refs = refs[2:]
        else:
            acc_out = refs[0]
            refs = refs[1:]
        (sidx0, sidx1, didx0, didx1, rows0, rows1, deg_v, acc_sh,
         sem0, sem1) = refs

        cid = lax.axis_index("c")
        sid = lax.axis_index("s")
        wid = sid * NC + cid
        ebase = wid * EPW

        _banded_copy(sid, lambda o, s: z_hbm.at[pl.ds(o, s)],
                     lambda o, s: acc_sh.at[pl.ds(o, s)])
        if with_deg:
            _zero_vmem_1d(deg_v, N)
        plsc.subcore_barrier()

        ones = jnp.ones((L,), jnp.float32)
        bufs = ((sidx0, didx0, rows0, sem0), (sidx1, didx1, rows1, sem1))

        def start(g, b):
            sidx, didx, rows, sem = bufs[b]
            base = ebase + g * CH
            pltpu.sync_copy(src_hbm.at[pl.ds(base, CH)], sidx)
            pltpu.sync_copy(dst_hbm.at[pl.ds(base, CH)], didx)
            return pltpu.async_copy(x_hbm.at[sidx], rows, sem)

        def finish(b):
            sidx, didx, rows, sem = bufs[b]
            pltpu.make_async_copy(x_hbm.at[sidx], rows, sem).wait()
            if with_deg:
                for i in range(CH // L):
                    d16 = didx[pl.ds(i * L, L)]
                    plsc.addupdate_scatter(deg_v, [d16], ones)
            pltpu.sync_copy(rows, acc_sh.at[didx], add=True)

        start(0, 0)

        def pair(p, _):
            start(2 * p + 1, 1)
            finish(0)

            @pl.when(2 * p + 2 < NCHUNK)
            def _():
                start(2 * p + 2, 0)
            finish(1)
            return 0

        lax.fori_loop(0, NCHUNK // 2, pair, 0)
        if NCHUNK % 2 == 1:
            finish(0)

        plsc.subcore_barrier()
        _banded_copy(sid, lambda o, s: acc_sh.at[pl.ds(o, s)],
                     lambda o, s: acc_out.at[cid, pl.ds(o, s), :])
        if with_deg:
            pltpu.sync_copy(deg_v, deg_out.at[wid])

    return k


_segsum_deg = _make_segsum(True)
_segsum = _make_segsum(False)


@functools.partial(
    pl.kernel, mesh=_MESH,
    out_type=(
        jax.ShapeDtypeStruct((E * H,), jnp.float32),
        jax.ShapeDtypeStruct((NW, N * H), jnp.float32),
    ),
    scratch_types=[
        pltpu.VMEM((N * H,), jnp.float32),
        pltpu.VMEM((N * H,), jnp.float32),
        pltpu.VMEM((N * H,), jnp.float32),
        pltpu.VMEM((CH,), jnp.int32),
        pltpu.VMEM((CH,), jnp.int32),
        pltpu.VMEM((CH * H,), jnp.float32),
    ],
    name="sc_gat_edge",
    compiler_params=pltpu.CompilerParams(needs_layout_passes=False),
)
def _gat_edge(als_hbm, ald_hbm, src_hbm, dst_hbm, ee_out, den_out,
              als_v, ald_v, den_v, sidx_v, didx_v, eec_v):
    cid = lax.axis_index("c")
    sid = lax.axis_index("s")
    wid = sid * NC + cid

    pltpu.sync_copy(als_hbm, als_v)
    pltpu.sync_copy(ald_hbm, ald_v)
    _zero_vmem_1d(den_v, N * H)

    iota = lax.broadcasted_iota(jnp.int32, (L,), 0)

    def chunk(g, _):
        base = wid * EPW + g * CH
        pltpu.sync_copy(src_hbm.at[pl.ds(base, CH)], sidx_v)
        pltpu.sync_copy(dst_hbm.at[pl.ds(base, CH)], didx_v)
        for i in range(CH // L):
            s16 = sidx_v[pl.ds(i * L, L)] * H
            d16 = didx_v[pl.ds(i * L, L)] * H
            for h in range(H):
                e = (plsc.load_gather(als_v, [s16 + h])
                     + plsc.load_gather(ald_v, [d16 + h]))
                e = jnp.where(e >= 0.0, e, 0.2 * e)
                ee = jnp.exp(e)
                plsc.store_scatter(eec_v, [iota * H + (i * L * H + h)], ee)
                plsc.addupdate_scatter(den_v, [d16 + h], ee)
        pltpu.sync_copy(eec_v, ee_out.at[pl.ds(base * H, CH * H)])
        return 0

    lax.fori_loop(0, NCHUNK, chunk, 0)
    pltpu.sync_copy(den_v, den_out.at[wid])


@functools.partial(
    pl.kernel, mesh=_MESH,
    out_type=jax.ShapeDtypeStruct((E * H,), jnp.float32),
    scratch_types=[
        pltpu.VMEM((N * H,), jnp.float32),
        pltpu.VMEM((CH,), jnp.int32),
        pltpu.VMEM((CH * H,), jnp.float32),
        pltpu.VMEM((CH * H,), jnp.float32),
    ],
    name="sc_gat_alpha",
    compiler_params=pltpu.CompilerParams(needs_layout_passes=False),
)
def _gat_alpha(ee_hbm, rden_hbm, dst_hbm, w_out, rden_v, didx_v, eec_v, wc_v):
    cid = lax.axis_index("c")
    sid = lax.axis_index("s")
    wid = sid * NC + cid

    pltpu.sync_copy(rden_hbm, rden_v)
    iota = lax.broadcasted_iota(jnp.int32, (L,), 0)

    def chunk(g, _):
        base = wid * EPW + g * CH
        pltpu.sync_copy(dst_hbm.at[pl.ds(base, CH)], didx_v)
        pltpu.sync_copy(ee_hbm.at[pl.ds(base * H, CH * H)], eec_v)
        for i in range(CH // L):
            d16 = didx_v[pl.ds(i * L, L)] * H
            for h in range(H):
                pos = iota * H + (i * L * H + h)
                ee16 = plsc.load_gather(eec_v, [pos])
                r16 = plsc.load_gather(rden_v, [d16 + h])
                plsc.store_scatter(wc_v, [pos], ee16 * r16)
        pltpu.sync_copy(wc_v, w_out.at[pl.ds(base * H, CH * H)])
        return 0

    lax.fori_loop(0, NCHUNK, chunk, 0)


CH3 = 40
NCHUNK3 = EPW // CH3

@functools.partial(
    pl.kernel, mesh=_MESH,
    out_type=jax.ShapeDtypeStruct((NC, N, D), jnp.float32),
    scratch_types=[
        pltpu.VMEM((CH3,), jnp.int32),
        pltpu.VMEM((CH3,), jnp.int32),
        pltpu.VMEM((CH3,), jnp.int32),
        pltpu.VMEM((CH3,), jnp.int32),
        pltpu.VMEM((CH3 * H,), jnp.float32),
        pltpu.VMEM((CH3 * H,), jnp.float32),
        pltpu.VMEM((CH3, H * D // 2), jnp.float32),
        pltpu.VMEM((CH3, H * D // 2), jnp.float32),
        pltpu.VMEM((CH3, D), jnp.float32),
        pltpu.VMEM_SHARED((N, D), jnp.float32),
        pltpu.SemaphoreType.DMA,
        pltpu.SemaphoreType.DMA,
    ],
    name="sc_gat_agg",
    compiler_params=pltpu.CompilerParams(needs_layout_passes=False),
)
def _gat_agg(hg_hbm, w_hbm, src_hbm, dst_hbm, z_hbm, acc_out,
             sidx0, sidx1, didx0, didx1, w0, w1, rows0, rows1, msg_v,
             acc_sh, sem0, sem1):
    cid = lax.axis_index("c")
    sid = lax.axis_index("s")
    wid = sid * NC + cid
    ebase = wid * EPW

    _banded_copy(sid, lambda o, s: z_hbm.at[pl.ds(o, s)],
                 lambda o, s: acc_sh.at[pl.ds(o, s)])
    plsc.subcore_barrier()

    bufs = ((sidx0, didx0, w0, rows0, sem0), (sidx1, didx1, w1, rows1, sem1))

    def start(g, b):
        sidx, didx, w_v, rows, sem = bufs[b]
        base = ebase + g * CH3
        pltpu.sync_copy(src_hbm.at[pl.ds(base, CH3)], sidx)
        pltpu.sync_copy(dst_hbm.at[pl.ds(base, CH3)], didx)
        pltpu.sync_copy(w_hbm.at[pl.ds(base * H, CH3 * H)], w_v)
        pltpu.async_copy(hg_hbm.at[sidx], rows, sem)

    def finish(b):
        sidx, didx, w_v, rows, sem = bufs[b]
        pltpu.make_async_copy(hg_hbm.at[sidx], rows, sem).wait()

        def edge4(q, _):
            w16 = w_v[pl.ds(q * (4 * H), 4 * H)]
            for k in range(4):
                i = q * 4 + k
                wb = [w16.at[jnp.broadcast_to(k * H + h, (L,))]
                      .get(mode="promise_in_bounds") for h in range(H)]
                for p in range(D // (2 * L)):
                    m0 = None
                    m1 = None
                    for h in range(H):
                        a, b = plsc.unpack(
                            plsc.bitcast(
                                rows[i, pl.ds(h * (D // 2) + p * L, L)],
                                jnp.bfloat16),
                            format=plsc.PackFormat.INTERLEAVED)
                        m0 = wb[h] * a if m0 is None else m0 + wb[h] * a
                        m1 = wb[h] * b if m1 is None else m1 + wb[h] * b
                    msg_v[i, pl.ds(2 * p * L, L)] = m0
                    msg_v[i, pl.ds((2 * p + 1) * L, L)] = m1
            return 0

        lax.fori_loop(0, CH3 // 4, edge4, 0)
        pltpu.sync_copy(msg_v, acc_sh.at[didx], add=True)

    start(0, 0)

    def pair(p, _):
        start(2 * p + 1, 1)
        finish(0)

        @pl.when(2 * p + 2 < NCHUNK3)
        def _():
            start(2 * p + 2, 0)
        finish(1)
        return 0

    lax.fori_loop(0, NCHUNK3 // 2, pair, 0)
    if NCHUNK3 % 2 == 1:
        finish(0)

    plsc.subcore_barrier()
    _banded_copy(sid, lambda o, s: acc_sh.at[pl.ds(o, s)],
                 lambda o, s: acc_out.at[cid, pl.ds(o, s), :])


RB = 1000
GRID = N // RB


def _ln(h, g, b):
    m = jnp.mean(h, axis=-1, keepdims=True)
    v = jnp.mean((h - m) * (h - m), axis=-1, keepdims=True)
    return (h - m) * jax.lax.rsqrt(v + 1e-5) * g + b


def _elu(h):
    return jnp.where(h > 0.0, h, jnp.exp(jnp.minimum(h, 0.0)) - 1.0)


def _tc_b_body(x_ref, acc_ref, degp_ref, W1l_ref, W1r_ref, b1_ref, g1_ref,
               bb1_ref, Wg_ref, as_ref, ad_ref,
               hg_ref, als_ref, ald_ref, deg_ref):
    deg = jnp.maximum(jnp.sum(degp_ref[...], axis=0), 1.0)
    deg_ref[...] = deg
    agg = (acc_ref[0] + acc_ref[1]) / deg
    s1 = agg @ W1l_ref[...] + x_ref[...] @ W1r_ref[...] + b1_ref[...][None]
    h1 = _elu(_ln(s1, g1_ref[...][None], bb1_ref[...][None]))
    hg = h1 @ Wg_ref[...]
    hg_ref[...] = hg.astype(jnp.bfloat16)
    als_ref[...] = hg @ as_ref[...]
    ald_ref[...] = hg @ ad_ref[...]


def _tc_b(x, acc, degp, W1l, W1r, b1, g1, bb1, Wg2d, As2, Ad2):
    return pl.pallas_call(
        _tc_b_body,
        grid=(GRID,),
        in_specs=[
            pl.BlockSpec((RB, D), lambda i: (i, 0)),
            pl.BlockSpec((NC, RB, D), lambda i: (0, i, 0)),
            pl.BlockSpec((NW, RB, 1), lambda i: (0, i, 0)),
            pl.BlockSpec((D, D), lambda i: (0, 0)),
            pl.BlockSpec((D, D), lambda i: (0, 0)),
            pl.BlockSpec((D,), lambda i: (0,)),
            pl.BlockSpec((D,), lambda i: (0,)),
            pl.BlockSpec((D,), lambda i: (0,)),
            pl.BlockSpec((D, H * D), lambda i: (0, 0)),
            pl.BlockSpec((H * D, H), lambda i: (0, 0)),
            pl.BlockSpec((H * D, H), lambda i: (0, 0)),
        ],
        out_specs=[
            pl.BlockSpec((RB, H * D), lambda i: (i, 0)),
            pl.BlockSpec((RB, H), lambda i: (i, 0)),
            pl.BlockSpec((RB, H), lambda i: (i, 0)),
            pl.BlockSpec((RB, 1), lambda i: (i, 0)),
        ],
        out_shape=[
            jax.ShapeDtypeStruct((N, H * D), jnp.bfloat16),
            jax.ShapeDtypeStruct((N, H), jnp.float32),
            jax.ShapeDtypeStruct((N, H), jnp.float32),
            jax.ShapeDtypeStruct((N, 1), jnp.float32),
        ],
    )(x, acc, degp, W1l, W1r, b1, g1, bb1, Wg2d, As2, Ad2)


def _tc_rden_body(den_ref, out_ref):
    s = jnp.sum(den_ref[...], axis=0)
    out_ref[...] = 1.0 / jnp.maximum(s, 1e-16)


def _tc_rden(denp):
    return pl.pallas_call(
        _tc_rden_body,
        grid=(GRID,),
        in_specs=[pl.BlockSpec((NW, RB, H), lambda i: (0, i, 0))],
        out_specs=pl.BlockSpec((RB, H), lambda i: (i, 0)),
        out_shape=jax.ShapeDtypeStruct((N, H), jnp.float32),
    )(denp)


def _tc_f_body(acc_ref, bg_ref, g2_ref, b2_ref, out_ref):
    gout = (acc_ref[0] + acc_ref[1]) * (1.0 / H) + bg_ref[...][None]
    out_ref[...] = _elu(_ln(gout, g2_ref[...][None], b2_ref[...][None]))


def _tc_f(acc, bg, g2, b2):
    return pl.pallas_call(
        _tc_f_body,
        grid=(GRID,),
        in_specs=[
            pl.BlockSpec((NC, RB, D), lambda i: (0, i, 0)),
            pl.BlockSpec((D,), lambda i: (0,)),
            pl.BlockSpec((D,), lambda i: (0,)),
            pl.BlockSpec((D,), lambda i: (0,)),
        ],
        out_specs=pl.BlockSpec((RB, D), lambda i: (i, 0)),
        out_shape=jax.ShapeDtypeStruct((N, D), jnp.float32),
    )(acc, bg, g2, b2)


def _tc_h_body(acc_ref, deg_ref, h2_ref, W3l_ref, W3r_ref, b3_ref,
               Wc1_ref, bc1_ref, Wc2_ref, bc2_ref, out_ref):
    agg = (acc_ref[0] + acc_ref[1]) / deg_ref[...]
    o3 = agg @ W3l_ref[...] + h2_ref[...] @ W3r_ref[...] + b3_ref[...][None]
    z = jnp.maximum(o3 @ Wc1_ref[...] + bc1_ref[...][None], 0.0)
    out_ref[...] = z @ Wc2_ref[...] + bc2_ref[...][None]


def _tc_h(acc, deg, h2, W3l, W3r, b3, Wc1, bc1, Wc2, bc2):
    return pl.pallas_call(
        _tc_h_body,
        grid=(GRID,),
        in_specs=[
            pl.BlockSpec((NC, RB, D), lambda i: (0, i, 0)),
            pl.BlockSpec((RB, 1), lambda i: (i, 0)),
            pl.BlockSpec((RB, D), lambda i: (i, 0)),
            pl.BlockSpec((D, D), lambda i: (0, 0)),
            pl.BlockSpec((D, D), lambda i: (0, 0)),
            pl.BlockSpec((D,), lambda i: (0,)),
            pl.BlockSpec((D, 64), lambda i: (0, 0)),
            pl.BlockSpec((64,), lambda i: (0,)),
            pl.BlockSpec((64, 2), lambda i: (0, 0)),
            pl.BlockSpec((2,), lambda i: (0,)),
        ],
        out_specs=pl.BlockSpec((RB, 2), lambda i: (i, 0)),
        out_shape=jax.ShapeDtypeStruct((N, 2), jnp.float32),
    )(acc, deg, h2, W3l, W3r, b3, Wc1, bc1, Wc2, bc2)


def kernel(x, edge_index, W1l, W1r, b1, ln1_g, ln1_b, Wg, a_src, a_dst, bg,
           ln2_g, ln2_b, W3l, W3r, b3, Wc1, bc1, Wc2, bc2):
    src = edge_index[0]
    dst = edge_index[1]
    zND = jnp.zeros((N, D), jnp.float32)

    eye = jnp.repeat(jnp.eye(H, dtype=jnp.float32), D, axis=0)
    As2 = eye * a_src.reshape(H * D)[:, None]
    Ad2 = eye * a_dst.reshape(H * D)[:, None]

    g = jnp.arange(H * D)
    w = g % D
    perm = (g // D) * D + (w // 32) * 32 + (w % 32) // 2 + (w % 2) * L
    Wg2d = Wg.reshape(D, H * D)[:, perm]
    As2 = As2[perm]
    Ad2 = Ad2[perm]

    acc1, degp = _segsum_deg(x, src, dst, zND)
    hg, als, ald, deg = _tc_b(x, acc1, degp.reshape(NW, N, 1), W1l, W1r, b1, ln1_g, ln1_b,
                              Wg2d, As2, Ad2)
    ee, denp = _gat_edge(als.reshape(N * H), ald.reshape(N * H), src, dst)
    rden = _tc_rden(denp.reshape(NW, N, H))
    w = _gat_alpha(ee, rden.reshape(N * H), dst)
    hgp = lax.bitcast_convert_type(hg.reshape(N, H * D // 2, 2), jnp.float32)
    gacc = _gat_agg(hgp, w, src, dst, zND)
    h2 = _tc_f(gacc, bg, ln2_g, ln2_b)
    acc3, = _segsum(h2, src, dst, zND)
    return _tc_h(acc3, deg, h2, W3l, W3r, b3, Wc1, bc1, Wc2, bc2)


# speedup vs baseline: 21.8669x; 1.0010x over previous
import functools
import jax
import jax.numpy as jnp
from jax import lax
from jax.experimental import pallas as pl
from jax.experimental.pallas import tpu as pltpu
from jax.experimental.pallas import tpu_sc as plsc

N = 10000
E = 320000
D = 128
H = 4

NC = 2
NS = 16
L = 16
NW = NC * NS
EPW = E // NW
CH = 80
NCHUNK = EPW // CH
_MESH = plsc.VectorSubcoreMesh(core_axis_name="c", subcore_axis_name="s")

_BAND = [(t * 624, 640 if t == NS - 1 else 624) for t in range(NS)]


def _banded_copy(sid, make_src, make_dst):
    for t, (off, sz) in enumerate(_BAND):
        @pl.when(sid == t)
        def _():
            pltpu.sync_copy(make_src(off, sz), make_dst(off, sz))


def _zero_vmem_1d(ref, n):
    z = jnp.zeros((L,), jnp.float32)

    def body(i, _):
        ref[pl.ds(i * L, L)] = z
        return 0

    lax.fori_loop(0, n // L, body, 0)


def _make_segsum(with_deg):
    outs = [jax.ShapeDtypeStruct((NC, N, D), jnp.float32)]
    if with_deg:
        outs.append(jax.ShapeDtypeStruct((NW, N), jnp.float32))

    scratch = [
        pltpu.VMEM((CH,), jnp.int32),
        pltpu.VMEM((CH,), jnp.int32),
        pltpu.VMEM((CH,), jnp.int32),
        pltpu.VMEM((CH,), jnp.int32),
        pltpu.VMEM((CH, D), jnp.float32),
        pltpu.VMEM((CH, D), jnp.float32),
        pltpu.VMEM((N,), jnp.float32),
        pltpu.VMEM_SHARED((N, D), jnp.float32),
        pltpu.SemaphoreType.DMA,
        pltpu.SemaphoreType.DMA,
    ]

    @functools.partial(
        pl.kernel, mesh=_MESH, out_type=tuple(outs), scratch_types=scratch,
        name="sc_segsum_deg" if with_deg else "sc_segsum",
        compiler_params=pltpu.CompilerParams(needs_layout_passes=False),
    )
    def k(x_hbm, src_hbm, dst_hbm, z_hbm, *refs):
        if with_deg:
            acc_out, deg_out = refs[0], refs[1]
            refs = refs[2:]
        else:
            acc_out = refs[0]
            refs = refs[1:]
        (sidx0, sidx1, didx0, didx1, rows0, rows1, deg_v, acc_sh,
         sem0, sem1) = refs

        cid = lax.axis_index("c")
        sid = lax.axis_index("s")
        wid = sid * NC + cid
        ebase = wid * EPW

        _banded_copy(sid, lambda o, s: z_hbm.at[pl.ds(o, s)],
                     lambda o, s: acc_sh.at[pl.ds(o, s)])
        if with_deg:
            _zero_vmem_1d(deg_v, N)
        plsc.subcore_barrier()

        ones = jnp.ones((L,), jnp.float32)
        bufs = ((sidx0, didx0, rows0, sem0), (sidx1, didx1, rows1, sem1))

        def start(g, b):
            sidx, didx, rows, sem = bufs[b]
            base = ebase + g * CH
            pltpu.sync_copy(src_hbm.at[pl.ds(base, CH)], sidx)
            pltpu.sync_copy(dst_hbm.at[pl.ds(base, CH)], didx)
            return pltpu.async_copy(x_hbm.at[sidx], rows, sem)

        def finish(b):
            sidx, didx, rows, sem = bufs[b]
            pltpu.make_async_copy(x_hbm.at[sidx], rows, sem).wait()
            if with_deg:
                for i in range(CH // L):
                    d16 = didx[pl.ds(i * L, L)]
                    plsc.addupdate_scatter(deg_v, [d16], ones)
            pltpu.sync_copy(rows, acc_sh.at[didx], add=True)

        start(0, 0)

        def pair(p, _):
            start(2 * p + 1, 1)
            finish(0)

            @pl.when(2 * p + 2 < NCHUNK)
            def _():
                start(2 * p + 2, 0)
            finish(1)
            return 0

        lax.fori_loop(0, NCHUNK // 2, pair, 0)
        if NCHUNK % 2 == 1:
            finish(0)

        plsc.subcore_barrier()
        _banded_copy(sid, lambda o, s: acc_sh.at[pl.ds(o, s)],
                     lambda o, s: acc_out.at[cid, pl.ds(o, s), :])
        if with_deg:
            pltpu.sync_copy(deg_v, deg_out.at[wid])

    return k


_segsum_deg = _make_segsum(True)
_segsum = _make_segsum(False)


@functools.partial(
    pl.kernel, mesh=_MESH,
    out_type=(
        jax.ShapeDtypeStruct((E * H,), jnp.float32),
        jax.ShapeDtypeStruct((NW, N * H), jnp.float32),
    ),
    scratch_types=[
        pltpu.VMEM((N * H,), jnp.float32),
        pltpu.VMEM((N * H,), jnp.float32),
        pltpu.VMEM((N * H,), jnp.float32),
        pltpu.VMEM((CH,), jnp.int32),
        pltpu.VMEM((CH,), jnp.int32),
        pltpu.VMEM((CH * H,), jnp.float32),
    ],
    name="sc_gat_edge",
    compiler_params=pltpu.CompilerParams(needs_layout_passes=False),
)
def _gat_edge(als_hbm, ald_hbm, src_hbm, dst_hbm, ee_out, den_out,
              als_v, ald_v, den_v, sidx_v, didx_v, eec_v):
    cid = lax.axis_index("c")
    sid = lax.axis_index("s")
    wid = sid * NC + cid

    pltpu.sync_copy(als_hbm, als_v)
    pltpu.sync_copy(ald_hbm, ald_v)
    _zero_vmem_1d(den_v, N * H)

    iota = lax.broadcasted_iota(jnp.int32, (L,), 0)

    def chunk(g, _):
        base = wid * EPW + g * CH
        pltpu.sync_copy(src_hbm.at[pl.ds(base, CH)], sidx_v)
        pltpu.sync_copy(dst_hbm.at[pl.ds(base, CH)], didx_v)
        for i in range(CH // L):
            s16 = sidx_v[pl.ds(i * L, L)] * H
            d16 = didx_v[pl.ds(i * L, L)] * H
            for h in range(H):
                e = (plsc.load_gather(als_v, [s16 + h])
                     + plsc.load_gather(ald_v, [d16 + h]))
                e = jnp.maximum(e, 0.2 * e)
                ee = jnp.exp(e)
                eec_v[pl.ds(h * CH + i * L, L)] = ee
                plsc.addupdate_scatter(den_v, [d16 + h], ee)
        pltpu.sync_copy(eec_v, ee_out.at[pl.ds(base * H, CH * H)])
        return 0

    lax.fori_loop(0, NCHUNK, chunk, 0)
    pltpu.sync_copy(den_v, den_out.at[wid])


@functools.partial(
    pl.kernel, mesh=_MESH,
    out_type=jax.ShapeDtypeStruct((E * H,), jnp.float32),
    scratch_types=[
        pltpu.VMEM((N * H,), jnp.float32),
        pltpu.VMEM((CH,), jnp.int32),
        pltpu.VMEM((CH * H,), jnp.float32),
        pltpu.VMEM((CH * H,), jnp.float32),
    ],
    name="sc_gat_alpha",
    compiler_params=pltpu.CompilerParams(needs_layout_passes=False),
)
def _gat_alpha(ee_hbm, rden_hbm, dst_hbm, w_out, rden_v, didx_v, eec_v, wc_v):
    cid = lax.axis_index("c")
    sid = lax.axis_index("s")
    wid = sid * NC + cid

    pltpu.sync_copy(rden_hbm, rden_v)
    iota = lax.broadcasted_iota(jnp.int32, (L,), 0)

    def chunk(g, _):
        base = wid * EPW + g * CH
        pltpu.sync_copy(dst_hbm.at[pl.ds(base, CH)], didx_v)
        pltpu.sync_copy(ee_hbm.at[pl.ds(base * H, CH * H)], eec_v)
        for i in range(CH // L):
            d16 = didx_v[pl.ds(i * L, L)] * H
            for h in range(H):
                ee16 = eec_v[pl.ds(h * CH + i * L, L)]
                r16 = plsc.load_gather(rden_v, [d16 + h])
                plsc.store_scatter(wc_v, [iota * H + (i * L * H + h)],
                                   ee16 * r16)
        pltpu.sync_copy(wc_v, w_out.at[pl.ds(base * H, CH * H)])
        return 0

    lax.fori_loop(0, NCHUNK, chunk, 0)


CH3 = 40
NCHUNK3 = EPW // CH3

@functools.partial(
    pl.kernel, mesh=_MESH,
    out_type=jax.ShapeDtypeStruct((NC, N, D), jnp.float32),
    scratch_types=[
        pltpu.VMEM((CH3,), jnp.int32),
        pltpu.VMEM((CH3,), jnp.int32),
        pltpu.VMEM((CH3,), jnp.int32),
        pltpu.VMEM((CH3,), jnp.int32),
        pltpu.VMEM((CH3 * H,), jnp.float32),
        pltpu.VMEM((CH3 * H,), jnp.float32),
        pltpu.VMEM((CH3, H * D // 2), jnp.float32),
        pltpu.VMEM((CH3, H * D // 2), jnp.float32),
        pltpu.VMEM((CH3, D), jnp.float32),
        pltpu.VMEM_SHARED((N, D), jnp.float32),
        pltpu.SemaphoreType.DMA,
        pltpu.SemaphoreType.DMA,
    ],
    name="sc_gat_agg",
    compiler_params=pltpu.CompilerParams(needs_layout_passes=False),
)
def _gat_agg(hg_hbm, w_hbm, src_hbm, dst_hbm, z_hbm, acc_out,
             sidx0, sidx1, didx0, didx1, w0, w1, rows0, rows1, msg_v,
             acc_sh, sem0, sem1):
    cid = lax.axis_index("c")
    sid = lax.axis_index("s")
    wid = sid * NC + cid
    ebase = wid * EPW

    _banded_copy(sid, lambda o, s: z_hbm.at[pl.ds(o, s)],
                 lambda o, s: acc_sh.at[pl.ds(o, s)])
    plsc.subcore_barrier()

    bufs = ((sidx0, didx0, w0, rows0, sem0), (sidx1, didx1, w1, rows1, sem1))

    def start(g, b):
        sidx, didx, w_v, rows, sem = bufs[b]
        base = ebase + g * CH3
        pltpu.sync_copy(src_hbm.at[pl.ds(base, CH3)], sidx)
        pltpu.sync_copy(dst_hbm.at[pl.ds(base, CH3)], didx)
        pltpu.sync_copy(w_hbm.at[pl.ds(base * H, CH3 * H)], w_v)
        pltpu.async_copy(hg_hbm.at[sidx], rows, sem)

    def finish(b):
        sidx, didx, w_v, rows, sem = bufs[b]
        pltpu.make_async_copy(hg_hbm.at[sidx], rows, sem).wait()

        def edge4(q, _):
            w16 = w_v[pl.ds(q * (4 * H), 4 * H)]
            for k in range(4):
                i = q * 4 + k
                wb = [w16.at[jnp.broadcast_to(k * H + h, (L,))]
                      .get(mode="promise_in_bounds") for h in range(H)]
                for p in range(D // (2 * L)):
                    m0 = None
                    m1 = None
                    for h in range(H):
                        a, b = plsc.unpack(
                            plsc.bitcast(
                                rows[i, pl.ds(h * (D // 2) + p * L, L)],
                                jnp.bfloat16),
                            format=plsc.PackFormat.INTERLEAVED)
                        m0 = wb[h] * a if m0 is None else m0 + wb[h] * a
                        m1 = wb[h] * b if m1 is None else m1 + wb[h] * b
                    msg_v[i, pl.ds(2 * p * L, L)] = m0
                    msg_v[i, pl.ds((2 * p + 1) * L, L)] = m1
            return 0

        lax.fori_loop(0, CH3 // 4, edge4, 0)
        pltpu.sync_copy(msg_v, acc_sh.at[didx], add=True)

    start(0, 0)

    def pair(p, _):
        start(2 * p + 1, 1)
        finish(0)

        @pl.when(2 * p + 2 < NCHUNK3)
        def _():
            start(2 * p + 2, 0)
        finish(1)
        return 0

    lax.fori_loop(0, NCHUNK3 // 2, pair, 0)
    if NCHUNK3 % 2 == 1:
        finish(0)

    plsc.subcore_barrier()
    _banded_copy(sid, lambda o, s: acc_sh.at[pl.ds(o, s)],
                 lambda o, s: acc_out.at[cid, pl.ds(o, s), :])


RB = 1000
GRID = N // RB


def _ln(h, g, b):
    m = jnp.mean(h, axis=-1, keepdims=True)
    v = jnp.mean((h - m) * (h - m), axis=-1, keepdims=True)
    return (h - m) * jax.lax.rsqrt(v + 1e-5) * g + b


def _elu(h):
    return jnp.where(h > 0.0, h, jnp.exp(jnp.minimum(h, 0.0)) - 1.0)


def _tc_b_body(x_ref, acc_ref, degp_ref, W1l_ref, W1r_ref, b1_ref, g1_ref,
               bb1_ref, Wg_ref, as_ref, ad_ref,
               hg_ref, als_ref, ald_ref, deg_ref):
    deg = jnp.maximum(jnp.sum(degp_ref[...], axis=0), 1.0)
    deg_ref[...] = deg
    agg = (acc_ref[0] + acc_ref[1]) / deg
    s1 = agg @ W1l_ref[...] + x_ref[...] @ W1r_ref[...] + b1_ref[...][None]
    h1 = _elu(_ln(s1, g1_ref[...][None], bb1_ref[...][None]))
    hg = h1 @ Wg_ref[...]
    hg_ref[...] = hg.astype(jnp.bfloat16)
    als_ref[...] = hg @ as_ref[...]
    ald_ref[...] = hg @ ad_ref[...]


def _tc_b(x, acc, degp, W1l, W1r, b1, g1, bb1, Wg2d, As2, Ad2):
    return pl.pallas_call(
        _tc_b_body,
        grid=(GRID,),
        in_specs=[
            pl.BlockSpec((RB, D), lambda i: (i, 0)),
            pl.BlockSpec((NC, RB, D), lambda i: (0, i, 0)),
            pl.BlockSpec((NW, RB, 1), lambda i: (0, i, 0)),
            pl.BlockSpec((D, D), lambda i: (0, 0)),
            pl.BlockSpec((D, D), lambda i: (0, 0)),
            pl.BlockSpec((D,), lambda i: (0,)),
            pl.BlockSpec((D,), lambda i: (0,)),
            pl.BlockSpec((D,), lambda i: (0,)),
            pl.BlockSpec((D, H * D), lambda i: (0, 0)),
            pl.BlockSpec((H * D, H), lambda i: (0, 0)),
            pl.BlockSpec((H * D, H), lambda i: (0, 0)),
        ],
        out_specs=[
            pl.BlockSpec((RB, H * D), lambda i: (i, 0)),
            pl.BlockSpec((RB, H), lambda i: (i, 0)),
            pl.BlockSpec((RB, H), lambda i: (i, 0)),
            pl.BlockSpec((RB, 1), lambda i: (i, 0)),
        ],
        out_shape=[
            jax.ShapeDtypeStruct((N, H * D), jnp.bfloat16),
            jax.ShapeDtypeStruct((N, H), jnp.float32),
            jax.ShapeDtypeStruct((N, H), jnp.float32),
            jax.ShapeDtypeStruct((N, 1), jnp.float32),
        ],
    )(x, acc, degp, W1l, W1r, b1, g1, bb1, Wg2d, As2, Ad2)


def _tc_rden_body(den_ref, out_ref):
    s = jnp.sum(den_ref[...], axis=0)
    out_ref[...] = 1.0 / jnp.maximum(s, 1e-16)


def _tc_rden(denp):
    return pl.pallas_call(
        _tc_rden_body,
        grid=(GRID,),
        in_specs=[pl.BlockSpec((NW, RB, H), lambda i: (0, i, 0))],
        out_specs=pl.BlockSpec((RB, H), lambda i: (i, 0)),
        out_shape=jax.ShapeDtypeStruct((N, H), jnp.float32),
    )(denp)


def _tc_f_body(acc_ref, bg_ref, g2_ref, b2_ref, out_ref):
    gout = (acc_ref[0] + acc_ref[1]) * (1.0 / H) + bg_ref[...][None]
    out_ref[...] = _elu(_ln(gout, g2_ref[...][None], b2_ref[...][None]))


def _tc_f(acc, bg, g2, b2):
    return pl.pallas_call(
        _tc_f_body,
        grid=(GRID,),
        in_specs=[
            pl.BlockSpec((NC, RB, D), lambda i: (0, i, 0)),
            pl.BlockSpec((D,), lambda i: (0,)),
            pl.BlockSpec((D,), lambda i: (0,)),
            pl.BlockSpec((D,), lambda i: (0,)),
        ],
        out_specs=pl.BlockSpec((RB, D), lambda i: (i, 0)),
        out_shape=jax.ShapeDtypeStruct((N, D), jnp.float32),
    )(acc, bg, g2, b2)


def _tc_h_body(acc_ref, deg_ref, h2_ref, W3l_ref, W3r_ref, b3_ref,
               Wc1_ref, bc1_ref, Wc2_ref, bc2_ref, out_ref):
    agg = (acc_ref[0] + acc_ref[1]) / deg_ref[...]
    o3 = agg @ W3l_ref[...] + h2_ref[...] @ W3r_ref[...] + b3_ref[...][None]
    z = jnp.maximum(o3 @ Wc1_ref[...] + bc1_ref[...][None], 0.0)
    out_ref[...] = z @ Wc2_ref[...] + bc2_ref[...][None]


def _tc_h(acc, deg, h2, W3l, W3r, b3, Wc1, bc1, Wc2, bc2):
    return pl.pallas_call(
        _tc_h_body,
        grid=(GRID,),
        in_specs=[
            pl.BlockSpec((NC, RB, D), lambda i: (0, i, 0)),
            pl.BlockSpec((RB, 1), lambda i: (i, 0)),
            pl.BlockSpec((RB, D), lambda i: (i, 0)),
            pl.BlockSpec((D, D), lambda i: (0, 0)),
            pl.BlockSpec((D, D), lambda i: (0, 0)),
            pl.BlockSpec((D,), lambda i: (0,)),
            pl.BlockSpec((D, 64), lambda i: (0, 0)),
            pl.BlockSpec((64,), lambda i: (0,)),
            pl.BlockSpec((64, 2), lambda i: (0, 0)),
            pl.BlockSpec((2,), lambda i: (0,)),
        ],
        out_specs=pl.BlockSpec((RB, 2), lambda i: (i, 0)),
        out_shape=jax.ShapeDtypeStruct((N, 2), jnp.float32),
    )(acc, deg, h2, W3l, W3r, b3, Wc1, bc1, Wc2, bc2)


def kernel(x, edge_index, W1l, W1r, b1, ln1_g, ln1_b, Wg, a_src, a_dst, bg,
           ln2_g, ln2_b, W3l, W3r, b3, Wc1, bc1, Wc2, bc2):
    src = edge_index[0]
    dst = edge_index[1]
    zND = jnp.zeros((N, D), jnp.float32)

    eye = jnp.repeat(jnp.eye(H, dtype=jnp.float32), D, axis=0)
    As2 = eye * a_src.reshape(H * D)[:, None]
    Ad2 = eye * a_dst.reshape(H * D)[:, None]

    g = jnp.arange(H * D)
    w = g % D
    perm = (g // D) * D + (w // 32) * 32 + (w % 32) // 2 + (w % 2) * L
    Wg2d = Wg.reshape(D, H * D)[:, perm]
    As2 = As2[perm]
    Ad2 = Ad2[perm]

    acc1, degp = _segsum_deg(x, src, dst, zND)
    hg, als, ald, deg = _tc_b(x, acc1, degp.reshape(NW, N, 1), W1l, W1r, b1, ln1_g, ln1_b,
                              Wg2d, As2, Ad2)
    ee, denp = _gat_edge(als.reshape(N * H), ald.reshape(N * H), src, dst)
    rden = _tc_rden(denp.reshape(NW, N, H))
    w = _gat_alpha(ee, rden.reshape(N * H), dst)
    hgp = lax.bitcast_convert_type(hg.reshape(N, H * D // 2, 2), jnp.float32)
    gacc = _gat_agg(hgp, w, src, dst, zND)
    h2 = _tc_f(gacc, bg, ln2_g, ln2_b)
    acc3, = _segsum(h2, src, dst, zND)
    return _tc_h(acc3, deg, h2, W3l, W3r, b3, Wc1, bc1, Wc2, bc2)
